# baseline (speedup 1.0000x reference)
import jax
import jax.numpy as jnp
from jax import lax
from jax.experimental import pallas as pl
from jax.experimental.pallas import tpu as pltpu
from jax.experimental.pallas import tpu_sc as plsc

_TOTAL = 16384
_B = 16
_NW = 32
_CHUNK = _TOTAL // _NW
_NV = _CHUNK // 16


_GATHER_DNUMS = lax.GatherDimensionNumbers(
    offset_dims=(), collapsed_slice_dims=(0,), start_index_map=(0,))


def _shuffle(x, idx):
    return lax.gather(x, idx[:, None], _GATHER_DNUMS, (1,),
                      mode=lax.GatherScatterMode.PROMISE_IN_BOUNDS)


def _allsum(x, lanes):
    for sh in (1, 2, 4, 8):
        x = x + _shuffle(x, lanes ^ sh)
    return x


def _prefix_sum(x, lanes, zf):
    for sh in (1, 2, 4, 8):
        shifted = _shuffle(x, jnp.maximum(lanes - sh, 0))
        x = x + jnp.where(lanes >= sh, shifted, zf)
    return x


def _sc_body(score_hbm, target_hbm, length_hbm, out_hbm,
             score_v, target_v, len_v, part_v):
    cid = lax.axis_index("c")
    sid = lax.axis_index("s")
    wid = cid * 16 + sid

    lanes = lax.iota(jnp.int32, 16)
    base = pl.multiple_of(wid * _CHUNK, 8)
    pltpu.sync_copy(length_hbm, len_v)
    pltpu.sync_copy(score_hbm.at[pl.ds(base, _CHUNK)], score_v)
    pltpu.sync_copy(target_hbm.at[pl.ds(base, _CHUNK)], target_v)

    zf = jnp.zeros((16,), jnp.float32)
    onesf = jnp.ones((16,), jnp.float32)
    cumf = _prefix_sum(len_v[...].astype(jnp.float32), lanes, zf)
    basef = base.astype(jnp.float32)
    s0 = _allsum(jnp.where(cumf <= basef, onesf, zf), lanes).astype(jnp.int32)
    s1 = jnp.minimum(s0 + 1, _B - 1)
    e0 = _allsum(jnp.where(lanes == s0, cumf, zf), lanes).astype(jnp.int32)

    def step(j, accs):
        a0t, a0s, a0d, a1t, a1s, a1d = accs
        off = j * 16
        t = target_v[pl.ds(off, 16)]
        s = score_v[pl.ds(off, 16)]
        idx = base + off + lanes
        te = jnp.exp(t)
        se = jnp.exp(s)
        d = te * s
        m0 = idx < e0
        m1 = idx >= e0
        a0t = a0t + jnp.where(m0, te, zf)
        a0s = a0s + jnp.where(m0, se, zf)
        a0d = a0d + jnp.where(m0, d, zf)
        a1t = a1t + jnp.where(m1, te, zf)
        a1s = a1s + jnp.where(m1, se, zf)
        a1d = a1d + jnp.where(m1, d, zf)
        return (a0t, a0s, a0d, a1t, a1s, a1d)

    accs = lax.fori_loop(0, _NV, step, (zf,) * 6)
    for q in range(3):
        tot0 = _allsum(accs[q], lanes)
        tot1 = _allsum(accs[3 + q], lanes)
        part_v[q, :] = (jnp.where(lanes == s0, tot0, zf)
                        + jnp.where(lanes == s1, tot1, zf))
    pltpu.sync_copy(part_v, out_hbm.at[wid])


def _tc_epilogue(parts_ref, out_ref):
    x = parts_ref[...]
    ts = jnp.sum(x[:, 0, :], axis=0)
    ss = jnp.sum(x[:, 1, :], axis=0)
    dd = jnp.sum(x[:, 2, :], axis=0)
    losses = jnp.log(ss) - dd / ts
    msk = jnp.abs(losses) > 0.0
    cnt = jnp.sum(msk.astype(jnp.float32))
    kept = jnp.sum(jnp.where(msk, losses, 0.0))
    res = jnp.where(cnt == 0.0, kept, kept / jnp.maximum(cnt, 1.0))
    out_ref[...] = jnp.broadcast_to(res, (1, 1))


@jax.jit
def kernel(score, target, length):
    mesh = plsc.VectorSubcoreMesh(core_axis_name="c", subcore_axis_name="s")
    parts = pl.kernel(
        _sc_body,
        out_type=jax.ShapeDtypeStruct((_NW, 3, 16), jnp.float32),
        mesh=mesh,
        scratch_types=[
            pltpu.VMEM((_CHUNK,), jnp.float32),
            pltpu.VMEM((_CHUNK,), jnp.float32),
            pltpu.VMEM((_B,), jnp.int32),
            pltpu.VMEM((3, 16), jnp.float32),
        ],
    )(score, target, length)
    res = pl.pallas_call(
        _tc_epilogue,
        out_shape=jax.ShapeDtypeStruct((1, 1), jnp.float32),
    )(parts)
    return res[0, 0]


# speedup vs baseline: 1.0414x; 1.0414x over previous
import jax
import jax.numpy as jnp
from jax import lax
from jax.experimental import pallas as pl
from jax.experimental.pallas import tpu as pltpu
from jax.experimental.pallas import tpu_sc as plsc

_TOTAL = 16384
_B = 16
_NW = 32
_CHUNK = _TOTAL // _NW
_NV = _CHUNK // 16


_GATHER_DNUMS = lax.GatherDimensionNumbers(
    offset_dims=(), collapsed_slice_dims=(0,), start_index_map=(0,))


def _shuffle(x, idx):
    return lax.gather(x, idx[:, None], _GATHER_DNUMS, (1,),
                      mode=lax.GatherScatterMode.PROMISE_IN_BOUNDS)


def _allsum(x, lanes):
    for sh in (1, 2, 4, 8):
        x = x + _shuffle(x, lanes ^ sh)
    return x


def _prefix_sum(x, lanes, zf):
    for sh in (1, 2, 4, 8):
        shifted = _shuffle(x, jnp.maximum(lanes - sh, 0))
        x = x + jnp.where(lanes >= sh, shifted, zf)
    return x


def _sc_body(score_hbm, target_hbm, length_hbm, out_hbm,
             score_v, target_v, len_v, part_v, lsem, ssem, tsem):
    cid = lax.axis_index("c")
    sid = lax.axis_index("s")
    wid = cid * 16 + sid

    lanes = lax.iota(jnp.int32, 16)
    base = pl.multiple_of(wid * _CHUNK, 8)
    cl = pltpu.async_copy(length_hbm, len_v, lsem)
    cs = pltpu.async_copy(score_hbm.at[pl.ds(base, _CHUNK)], score_v, ssem)
    ct = pltpu.async_copy(target_hbm.at[pl.ds(base, _CHUNK)], target_v, tsem)
    cl.wait()

    zf = jnp.zeros((16,), jnp.float32)
    onesf = jnp.ones((16,), jnp.float32)
    cumf = _prefix_sum(len_v[...].astype(jnp.float32), lanes, zf)
    basef = base.astype(jnp.float32)
    s0 = _allsum(jnp.where(cumf <= basef, onesf, zf), lanes).astype(jnp.int32)
    s1 = jnp.minimum(s0 + 1, _B - 1)
    e0 = _allsum(jnp.where(lanes == s0, cumf, zf), lanes).astype(jnp.int32)

    def step(j, accs):
        a0t, a0s, a0d, a1t, a1s, a1d = accs
        off = j * 16
        t = target_v[pl.ds(off, 16)]
        s = score_v[pl.ds(off, 16)]
        idx = base + off + lanes
        te = jnp.exp(t)
        se = jnp.exp(s)
        d = te * s
        m0 = idx < e0
        m1 = idx >= e0
        a0t = a0t + jnp.where(m0, te, zf)
        a0s = a0s + jnp.where(m0, se, zf)
        a0d = a0d + jnp.where(m0, d, zf)
        a1t = a1t + jnp.where(m1, te, zf)
        a1s = a1s + jnp.where(m1, se, zf)
        a1d = a1d + jnp.where(m1, d, zf)
        return (a0t, a0s, a0d, a1t, a1s, a1d)

    cs.wait()
    ct.wait()
    accs = lax.fori_loop(0, _NV, step, (zf,) * 6, unroll=4)
    for q in range(3):
        tot0 = _allsum(accs[q], lanes)
        tot1 = _allsum(accs[3 + q], lanes)
        part_v[q, :] = (jnp.where(lanes == s0, tot0, zf)
                        + jnp.where(lanes == s1, tot1, zf))
    pltpu.sync_copy(part_v, out_hbm.at[wid])


def _tc_epilogue(parts_ref, out_ref):
    x = parts_ref[...]
    ts = jnp.sum(x[:, 0, :], axis=0)
    ss = jnp.sum(x[:, 1, :], axis=0)
    dd = jnp.sum(x[:, 2, :], axis=0)
    losses = jnp.log(ss) - dd / ts
    msk = jnp.abs(losses) > 0.0
    cnt = jnp.sum(msk.astype(jnp.float32))
    kept = jnp.sum(jnp.where(msk, losses, 0.0))
    res = jnp.where(cnt == 0.0, kept, kept / jnp.maximum(cnt, 1.0))
    out_ref[...] = jnp.broadcast_to(res, (1, 1))


@jax.jit
def kernel(score, target, length):
    mesh = plsc.VectorSubcoreMesh(core_axis_name="c", subcore_axis_name="s")
    parts = pl.kernel(
        _sc_body,
        out_type=jax.ShapeDtypeStruct((_NW, 3, 16), jnp.float32),
        mesh=mesh,
        scratch_types=[
            pltpu.VMEM((_CHUNK,), jnp.float32),
            pltpu.VMEM((_CHUNK,), jnp.float32),
            pltpu.VMEM((_B,), jnp.int32),
            pltpu.VMEM((3, 16), jnp.float32),
            pltpu.SemaphoreType.DMA,
            pltpu.SemaphoreType.DMA,
            pltpu.SemaphoreType.DMA,
        ],
    )(score, target, length)
    res = pl.pallas_call(
        _tc_epilogue,
        out_shape=jax.ShapeDtypeStruct((1, 1), jnp.float32),
    )(parts)
    return res[0, 0]
